# baseline (speedup 1.0000x reference)
import jax
import jax.numpy as jnp
from jax import lax
from jax.experimental import pallas as pl
from jax.experimental.pallas import tpu as pltpu

B, N, D, DO = 256, 64, 16, 16
BB = 8


def _tc_body(v_ref, f_ref, a_ref, w12_ref, sxy_ref, w3_ref, b_ref, o_ref):
    ii = lax.broadcasted_iota(jnp.int32, (N, N), 0)
    jj = lax.broadcasted_iota(jnp.int32, (N, N), 1)
    eye = (ii == jj).astype(jnp.float32)
    rr = lax.broadcasted_iota(jnp.int32, (BB * N, BB * DO), 0)
    cc = lax.broadcasted_iota(jnp.int32, (BB * N, BB * DO), 1)
    bdmask = ((cc // DO) == (rr // N)).astype(jnp.float32)

    w30 = w3_ref[0:1, :]
    w31 = w3_ref[1:2, :]
    bias = b_ref[...]

    vflat = v_ref[...].reshape(BB * N, 2 * N)
    vxy = jnp.dot(vflat, sxy_ref[...])
    fflat = f_ref[...].reshape(BB * N, D)
    gb = jnp.dot(fflat, w12_ref[...])
    g_all = gb[:, :DO]
    astack = jnp.concatenate([a_ref[t] for t in range(BB)], axis=1)
    gbd = jnp.concatenate([g_all] * BB, axis=1) * bdmask
    s_all = jnp.dot(astack, gbd)

    for t in range(BB):
        a = a_ref[t]
        vx = vxy[t * N:(t + 1) * N, :N]
        vy = vxy[t * N:(t + 1) * N, N:]
        dx = jnp.sum(a * vx, axis=1, keepdims=True)
        dy = jnp.sum(a * vy, axis=1, keepdims=True)
        aii = jnp.sum(a * eye, axis=1, keepdims=True)
        vdx = jnp.sum(vx * eye, axis=1, keepdims=True)
        vdy = jnp.sum(vy * eye, axis=1, keepdims=True)
        g_t = gb[t * N:(t + 1) * N, :DO]
        base_t = gb[t * N:(t + 1) * N, DO:]
        s_t = s_all[:, t * DO:(t + 1) * DO]
        dcon = dx * w30 + dy * w31
        selfd = vdx * w30 + vdy * w31
        out = base_t + bias + s_t + dcon - aii * (g_t + selfd)
        o_ref[t] = jnp.maximum(out, 0.0)


@jax.jit
def _run_tc(v, f, a, w12, sxy, w3, bs):
    grid = (B // BB,)
    out = pl.pallas_call(
        _tc_body,
        grid=grid,
        in_specs=[
            pl.BlockSpec((BB, N, 2 * N), lambda p: (p, 0, 0)),
            pl.BlockSpec((BB, N, D), lambda p: (p, 0, 0)),
            pl.BlockSpec((BB, N, N), lambda p: (p, 0, 0)),
            pl.BlockSpec((D, 2 * DO), lambda p: (0, 0)),
            pl.BlockSpec((2 * N, 2 * N), lambda p: (0, 0)),
            pl.BlockSpec((2, DO), lambda p: (0, 0)),
            pl.BlockSpec((1, DO), lambda p: (0, 0)),
        ],
        out_specs=pl.BlockSpec((BB, N, DO), lambda p: (p, 0, 0)),
        out_shape=jax.ShapeDtypeStruct((B, N, DO), jnp.float32),
        compiler_params=pltpu.CompilerParams(
            dimension_semantics=("arbitrary",),
        ),
    )(v, f, a, w12, sxy, w3, bs)
    return out


def kernel(diff_vecs, agent_features, A, W, b):
    v = diff_vecs.reshape(B, N, 2 * N)
    w1 = (N - 1.0) * W[:, :D].T
    w2 = W[:, D:2 * D].T
    w12 = jnp.concatenate([w2, w1], axis=1)
    w3 = W[:, 2 * D:].T
    bs = ((N - 1.0) * b).reshape(1, DO)
    l_i = jnp.arange(2 * N)[:, None]
    j_i = jnp.arange(N)[None, :]
    sx = (l_i == 2 * j_i).astype(jnp.float32)
    sy = (l_i == 2 * j_i + 1).astype(jnp.float32)
    sxy = jnp.concatenate([sx, sy], axis=1)
    out = _run_tc(v, agent_features, A, w12, sxy, w3, bs)
    return (diff_vecs, out)


# speedup vs baseline: 1.8036x; 1.8036x over previous
import jax
import jax.numpy as jnp
from jax.experimental import pallas as pl
from jax.experimental.pallas import tpu as pltpu

B, N, D, DO = 256, 64, 16, 16
BB = 8
R = BB * N


def _tc_body(v_ref, f_ref, a_ref, w12_ref, dup_ref, w3r_ref, eyed_ref,
             eyet_ref, b_ref, o_ref):
    a3 = a_ref[...]
    a2 = a3.reshape(R, N)
    vflat = v_ref[...].reshape(R, 2 * N)
    f2 = f_ref[...].reshape(R, D)

    gb = jnp.dot(f2, w12_ref[...])
    g_all = gb[:, :DO]

    s_all = jnp.concatenate(
        [jnp.dot(a3[t], g_all[t * N:(t + 1) * N, :]) for t in range(BB)],
        axis=0)

    adup = jnp.dot(a2, dup_ref[...])
    dcon = jnp.dot(adup * vflat, w3r_ref[...])

    selfd = jnp.dot(eyed_ref[...] * vflat, w3r_ref[...])
    ones = jnp.ones((N, DO), jnp.float32)
    aii = jnp.dot(eyet_ref[...] * a2, ones)

    out = gb[:, DO:] + b_ref[...] + s_all + dcon - aii * (g_all + selfd)
    o_ref[...] = jnp.maximum(out, 0.0).reshape(BB, N, DO)


@jax.jit
def _run_tc(v, f, a, w12, dup, w3rep, eyed, eyet, bs):
    grid = (B // BB,)
    out = pl.pallas_call(
        _tc_body,
        grid=grid,
        in_specs=[
            pl.BlockSpec((BB, N, 2 * N), lambda p: (p, 0, 0)),
            pl.BlockSpec((BB, N, D), lambda p: (p, 0, 0)),
            pl.BlockSpec((BB, N, N), lambda p: (p, 0, 0)),
            pl.BlockSpec((D, 2 * DO), lambda p: (0, 0)),
            pl.BlockSpec((N, 2 * N), lambda p: (0, 0)),
            pl.BlockSpec((2 * N, DO), lambda p: (0, 0)),
            pl.BlockSpec((R, 2 * N), lambda p: (0, 0)),
            pl.BlockSpec((R, N), lambda p: (0, 0)),
            pl.BlockSpec((1, DO), lambda p: (0, 0)),
        ],
        out_specs=pl.BlockSpec((BB, N, DO), lambda p: (p, 0, 0)),
        out_shape=jax.ShapeDtypeStruct((B, N, DO), jnp.float32),
        compiler_params=pltpu.CompilerParams(
            dimension_semantics=("arbitrary",),
        ),
    )(v, f, a, w12, dup, w3rep, eyed, eyet, bs)
    return out


def kernel(diff_vecs, agent_features, A, W, b):
    v = diff_vecs.reshape(B, N, 2 * N)
    w1 = (N - 1.0) * W[:, :D].T
    w2 = W[:, D:2 * D].T
    w12 = jnp.concatenate([w2, w1], axis=1)
    w3rep = jnp.tile(W[:, 2 * D:].T, (N, 1))
    bs = ((N - 1.0) * b).reshape(1, DO)

    jr = jnp.arange(N)
    lr = jnp.arange(2 * N)
    rr = jnp.arange(R)
    dup = (lr[None, :] // 2 == jr[:, None]).astype(jnp.float32)
    eyed = (lr[None, :] // 2 == (rr[:, None] % N)).astype(jnp.float32)
    eyet = (jr[None, :] == (rr[:, None] % N)).astype(jnp.float32)

    out = _run_tc(v, agent_features, A, w12, dup, w3rep, eyed, eyet, bs)
    return (diff_vecs, out)


# speedup vs baseline: 3.3046x; 1.8322x over previous
import jax
import jax.numpy as jnp
from jax import lax
from jax.experimental import pallas as pl
from jax.experimental.pallas import tpu as pltpu

B, N, D, DO = 256, 64, 16, 16
IB = 8


def _g_body(f3_ref, w2_ref, g_ref):
    for o in range(DO):
        acc = w2_ref[o, 0] * f3_ref[0]
        for d in range(1, D):
            acc = acc + w2_ref[o, d] * f3_ref[d]
        g_ref[o] = acc


def _main_body(a_ref, dx_ref, dy_ref, f_ref, gt_ref, g_ref, w1_ref, c_ref,
               o_ref):
    p = pl.program_id(0)
    a = a_ref[...]
    dxv = dx_ref[...]
    dyv = dy_ref[...]
    jj = lax.broadcasted_iota(jnp.int32, (IB, N, B), 1)
    rr = lax.broadcasted_iota(jnp.int32, (IB, N, B), 0)
    msk = (jj == p * IB + rr).astype(jnp.float32)

    dxw = jnp.sum(a * dxv, axis=1)
    dyw = jnp.sum(a * dyv, axis=1)
    aii = jnp.sum(a * msk, axis=1)
    vdx = jnp.sum(dxv * msk, axis=1)
    vdy = jnp.sum(dyv * msk, axis=1)

    g = g_ref[...]
    w30 = c_ref[:, 0:1]
    w31 = c_ref[:, 1:2]
    bs = c_ref[:, 2:3]
    for r in range(IB):
        s = jnp.sum(g * a[r][None], axis=1)
        base = jnp.dot(w1_ref[...], f_ref[r])
        g_i = gt_ref[r]
        dcon = w30 * dxw[r][None] + w31 * dyw[r][None]
        selfd = g_i + w30 * vdx[r][None] + w31 * vdy[r][None]
        out = base + bs + s + dcon - aii[r][None] * selfd
        o_ref[r] = jnp.maximum(out, 0.0)


@jax.jit
def _run(at, dxt, dyt, ft, f3, w1s, w2, consts):
    g = pl.pallas_call(
        _g_body,
        in_specs=[
            pl.BlockSpec((D, N, B), lambda: (0, 0, 0)),
            pl.BlockSpec((DO, D), lambda: (0, 0)),
        ],
        out_specs=pl.BlockSpec((DO, N, B), lambda: (0, 0, 0)),
        out_shape=jax.ShapeDtypeStruct((DO, N, B), jnp.float32),
    )(f3, w2)
    gt = jnp.transpose(g, (1, 0, 2))
    out = pl.pallas_call(
        _main_body,
        grid=(N // IB,),
        in_specs=[
            pl.BlockSpec((IB, N, B), lambda p: (p, 0, 0)),
            pl.BlockSpec((IB, N, B), lambda p: (p, 0, 0)),
            pl.BlockSpec((IB, N, B), lambda p: (p, 0, 0)),
            pl.BlockSpec((IB, D, B), lambda p: (p, 0, 0)),
            pl.BlockSpec((IB, DO, B), lambda p: (p, 0, 0)),
            pl.BlockSpec((DO, N, B), lambda p: (0, 0, 0)),
            pl.BlockSpec((DO, D), lambda p: (0, 0)),
            pl.BlockSpec((DO, 128), lambda p: (0, 0)),
        ],
        out_specs=pl.BlockSpec((IB, DO, B), lambda p: (p, 0, 0)),
        out_shape=jax.ShapeDtypeStruct((N, DO, B), jnp.float32),
        compiler_params=pltpu.CompilerParams(
            dimension_semantics=("arbitrary",),
        ),
    )(at, dxt, dyt, ft, gt, g, w1s, consts)
    return out


def kernel(diff_vecs, agent_features, A, W, b):
    at = jnp.transpose(A, (1, 2, 0))
    dxt = jnp.transpose(diff_vecs[..., 0], (1, 2, 0))
    dyt = jnp.transpose(diff_vecs[..., 1], (1, 2, 0))
    ft = jnp.transpose(agent_features, (1, 2, 0))
    f3 = jnp.transpose(agent_features, (2, 1, 0))
    w1s = (N - 1.0) * W[:, :D]
    w2 = W[:, D:2 * D]
    consts = jnp.zeros((DO, 128), jnp.float32)
    consts = consts.at[:, 0].set(W[:, 2 * D])
    consts = consts.at[:, 1].set(W[:, 2 * D + 1])
    consts = consts.at[:, 2].set((N - 1.0) * b)
    out_t = _run(at, dxt, dyt, ft, f3, w1s, w2, consts)
    out = jnp.transpose(out_t, (2, 0, 1))
    return (diff_vecs, out)


# speedup vs baseline: 3.4654x; 1.0487x over previous
import jax
import jax.numpy as jnp
from jax import lax
from jax.experimental import pallas as pl
from jax.experimental.pallas import tpu as pltpu

B, N, D, DO = 256, 64, 16, 16
IB = 8


def _g_body(f3_ref, w2_ref, g_ref):
    for o in range(DO):
        acc = w2_ref[o, 0] * f3_ref[0]
        for d in range(1, D):
            acc = acc + w2_ref[o, d] * f3_ref[d]
        g_ref[o] = acc


def _main_body(a_ref, d4_ref, f_ref, gt_ref, g_ref, w1_ref, c_ref,
               o_ref):
    p = pl.program_id(0)
    a = a_ref[...]
    dxv = d4_ref[:, :, 0, :]
    dyv = d4_ref[:, :, 1, :]
    jj = lax.broadcasted_iota(jnp.int32, (IB, N, B), 1)
    rr = lax.broadcasted_iota(jnp.int32, (IB, N, B), 0)
    msk = (jj == p * IB + rr).astype(jnp.float32)

    dxw = jnp.sum(a * dxv, axis=1)
    dyw = jnp.sum(a * dyv, axis=1)
    aii = jnp.sum(a * msk, axis=1)
    vdx = jnp.sum(dxv * msk, axis=1)
    vdy = jnp.sum(dyv * msk, axis=1)

    g = g_ref[...]
    w30 = c_ref[:, 0:1]
    w31 = c_ref[:, 1:2]
    bs = c_ref[:, 2:3]
    for r in range(IB):
        s = jnp.sum(g * a[r][None], axis=1)
        base = jnp.dot(w1_ref[...], f_ref[r])
        g_i = gt_ref[r]
        dcon = w30 * dxw[r][None] + w31 * dyw[r][None]
        selfd = g_i + w30 * vdx[r][None] + w31 * vdy[r][None]
        out = base + bs + s + dcon - aii[r][None] * selfd
        o_ref[r] = jnp.maximum(out, 0.0)


@jax.jit
def _run(at, d4, ft, f3, w1s, w2, consts):
    g = pl.pallas_call(
        _g_body,
        in_specs=[
            pl.BlockSpec((D, N, B), lambda: (0, 0, 0)),
            pl.BlockSpec((DO, D), lambda: (0, 0)),
        ],
        out_specs=pl.BlockSpec((DO, N, B), lambda: (0, 0, 0)),
        out_shape=jax.ShapeDtypeStruct((DO, N, B), jnp.float32),
    )(f3, w2)
    gt = jnp.transpose(g, (1, 0, 2))
    out = pl.pallas_call(
        _main_body,
        grid=(N // IB,),
        in_specs=[
            pl.BlockSpec((IB, N, B), lambda p: (p, 0, 0)),
            pl.BlockSpec((IB, N, 2, B), lambda p: (p, 0, 0, 0)),
            pl.BlockSpec((IB, D, B), lambda p: (p, 0, 0)),
            pl.BlockSpec((IB, DO, B), lambda p: (p, 0, 0)),
            pl.BlockSpec((DO, N, B), lambda p: (0, 0, 0)),
            pl.BlockSpec((DO, D), lambda p: (0, 0)),
            pl.BlockSpec((DO, 128), lambda p: (0, 0)),
        ],
        out_specs=pl.BlockSpec((IB, DO, B), lambda p: (p, 0, 0)),
        out_shape=jax.ShapeDtypeStruct((N, DO, B), jnp.float32),
        compiler_params=pltpu.CompilerParams(
            dimension_semantics=("arbitrary",),
        ),
    )(at, d4, ft, gt, g, w1s, consts)
    return out


def kernel(diff_vecs, agent_features, A, W, b):
    at = jnp.transpose(A, (1, 2, 0))
    d4 = jnp.transpose(diff_vecs, (1, 2, 3, 0))
    ft = jnp.transpose(agent_features, (1, 2, 0))
    f3 = jnp.transpose(agent_features, (2, 1, 0))
    w1s = (N - 1.0) * W[:, :D]
    w2 = W[:, D:2 * D]
    consts = jnp.zeros((DO, 128), jnp.float32)
    consts = consts.at[:, 0].set(W[:, 2 * D])
    consts = consts.at[:, 1].set(W[:, 2 * D + 1])
    consts = consts.at[:, 2].set((N - 1.0) * b)
    out_t = _run(at, d4, ft, f3, w1s, w2, consts)
    out = jnp.transpose(out_t, (2, 0, 1))
    return (diff_vecs, out)


# speedup vs baseline: 4.0754x; 1.1760x over previous
import jax
import jax.numpy as jnp
from jax import lax
from jax.experimental import pallas as pl
from jax.experimental.pallas import tpu as pltpu

B, N, D, DO = 256, 64, 16, 16
IB = 8


def _g_body(f3_ref, w2_ref, g_ref):
    for o in range(DO):
        acc = w2_ref[o, 0] * f3_ref[0]
        for d in range(1, D):
            acc = acc + w2_ref[o, d] * f3_ref[d]
        g_ref[o] = acc


def _main_body(a_ref, d4_ref, ad_ref, dd_ref, f_ref, gt_ref, g_ref, w1_ref,
               c_ref, o_ref):
    a = a_ref[...]
    dxv = d4_ref[:, :, 0, :]
    dyv = d4_ref[:, :, 1, :]

    dxw = jnp.sum(a * dxv, axis=1)
    dyw = jnp.sum(a * dyv, axis=1)

    g = g_ref[...]
    w30 = c_ref[:, 0:1]
    w31 = c_ref[:, 1:2]
    bs = c_ref[:, 2:3]
    for r in range(IB):
        s = jnp.sum(g * a[r][None], axis=1)
        base = jnp.dot(w1_ref[...], f_ref[r])
        g_i = gt_ref[r]
        aii = ad_ref[r, r]
        vdx = dd_ref[r, r, 0]
        vdy = dd_ref[r, r, 1]
        dcon = w30 * dxw[r][None] + w31 * dyw[r][None]
        selfd = g_i + w30 * vdx[None] + w31 * vdy[None]
        out = base + bs + s + dcon - aii[None] * selfd
        o_ref[r] = jnp.maximum(out, 0.0)


@jax.jit
def _run(at, d4, ft, f3, w1s, w2, consts):
    g = pl.pallas_call(
        _g_body,
        in_specs=[
            pl.BlockSpec((D, N, B), lambda: (0, 0, 0)),
            pl.BlockSpec((DO, D), lambda: (0, 0)),
        ],
        out_specs=pl.BlockSpec((DO, N, B), lambda: (0, 0, 0)),
        out_shape=jax.ShapeDtypeStruct((DO, N, B), jnp.float32),
    )(f3, w2)
    gt = jnp.transpose(g, (1, 0, 2))
    out = pl.pallas_call(
        _main_body,
        grid=(N // IB,),
        in_specs=[
            pl.BlockSpec((IB, N, B), lambda p: (p, 0, 0)),
            pl.BlockSpec((IB, N, 2, B), lambda p: (p, 0, 0, 0)),
            pl.BlockSpec((IB, IB, B), lambda p: (p, p, 0)),
            pl.BlockSpec((IB, IB, 2, B), lambda p: (p, p, 0, 0)),
            pl.BlockSpec((IB, D, B), lambda p: (p, 0, 0)),
            pl.BlockSpec((IB, DO, B), lambda p: (p, 0, 0)),
            pl.BlockSpec((DO, N, B), lambda p: (0, 0, 0)),
            pl.BlockSpec((DO, D), lambda p: (0, 0)),
            pl.BlockSpec((DO, 128), lambda p: (0, 0)),
        ],
        out_specs=pl.BlockSpec((IB, DO, B), lambda p: (p, 0, 0)),
        out_shape=jax.ShapeDtypeStruct((N, DO, B), jnp.float32),
        compiler_params=pltpu.CompilerParams(
            dimension_semantics=("arbitrary",),
        ),
    )(at, d4, at, d4, ft, gt, g, w1s, consts)
    return out


def kernel(diff_vecs, agent_features, A, W, b):
    at = jnp.transpose(A, (1, 2, 0))
    d4 = jnp.transpose(diff_vecs, (1, 2, 3, 0))
    ft = jnp.transpose(agent_features, (1, 2, 0))
    f3 = jnp.transpose(agent_features, (2, 1, 0))
    w1s = (N - 1.0) * W[:, :D]
    w2 = W[:, D:2 * D]
    consts = jnp.zeros((DO, 128), jnp.float32)
    consts = consts.at[:, 0].set(W[:, 2 * D])
    consts = consts.at[:, 1].set(W[:, 2 * D + 1])
    consts = consts.at[:, 2].set((N - 1.0) * b)
    out_t = _run(at, d4, ft, f3, w1s, w2, consts)
    out = jnp.transpose(out_t, (2, 0, 1))
    return (diff_vecs, out)


# speedup vs baseline: 8.0031x; 1.9637x over previous
import jax
import jax.numpy as jnp
from jax.experimental import pallas as pl
from jax.experimental.pallas import tpu as pltpu

B, N, D, DO = 256, 64, 16, 16
IB = 8


def _g_body(f_ref, w_ref, g_ref, gt_ref):
    w2 = w_ref[:, D:2 * D]
    for j in range(N):
        r = jnp.dot(w2, f_ref[j])
        gt_ref[j] = r
        g_ref[:, j, :] = r


def _main_body(a_ref, d4_ref, ad_ref, dd_ref, f_ref, gt_ref, g_ref, w_ref,
               b_ref, o_ref, o2_ref):
    a = a_ref[...]
    d4 = d4_ref[...]
    o2_ref[...] = d4
    dxv = d4[:, :, 0, :]
    dyv = d4[:, :, 1, :]

    dxw = jnp.sum(a * dxv, axis=1)
    dyw = jnp.sum(a * dyv, axis=1)

    g = g_ref[...]
    w1 = w_ref[:, :D]
    w30 = w_ref[:, 2 * D:2 * D + 1]
    w31 = w_ref[:, 2 * D + 1:2 * D + 2]
    bs = (N - 1.0) * jnp.transpose(b_ref[...], (1, 0))
    for r in range(IB):
        s = jnp.sum(g * a[r][None], axis=1)
        base = (N - 1.0) * jnp.dot(w1, f_ref[r])
        g_i = gt_ref[r]
        aii = ad_ref[r, r]
        vdx = dd_ref[r, r, 0]
        vdy = dd_ref[r, r, 1]
        dcon = w30 * dxw[r][None] + w31 * dyw[r][None]
        selfd = g_i + w30 * vdx[None] + w31 * vdy[None]
        out = base + bs + s + dcon - aii[None] * selfd
        o_ref[r] = jnp.maximum(out, 0.0)


@jax.jit
def _run(at, d4, ft, w, b2):
    g, gt = pl.pallas_call(
        _g_body,
        in_specs=[
            pl.BlockSpec((N, D, B), lambda: (0, 0, 0)),
            pl.BlockSpec((DO, 2 * D + 2), lambda: (0, 0)),
        ],
        out_specs=[
            pl.BlockSpec((DO, N, B), lambda: (0, 0, 0)),
            pl.BlockSpec((N, DO, B), lambda: (0, 0, 0)),
        ],
        out_shape=[
            jax.ShapeDtypeStruct((DO, N, B), jnp.float32),
            jax.ShapeDtypeStruct((N, DO, B), jnp.float32),
        ],
    )(ft, w)
    out, d4c = pl.pallas_call(
        _main_body,
        grid=(N // IB,),
        in_specs=[
            pl.BlockSpec((IB, N, B), lambda p: (p, 0, 0)),
            pl.BlockSpec((IB, N, 2, B), lambda p: (p, 0, 0, 0)),
            pl.BlockSpec((IB, IB, B), lambda p: (p, p, 0)),
            pl.BlockSpec((IB, IB, 2, B), lambda p: (p, p, 0, 0)),
            pl.BlockSpec((IB, D, B), lambda p: (p, 0, 0)),
            pl.BlockSpec((IB, DO, B), lambda p: (p, 0, 0)),
            pl.BlockSpec((DO, N, B), lambda p: (0, 0, 0)),
            pl.BlockSpec((DO, 2 * D + 2), lambda p: (0, 0)),
            pl.BlockSpec((1, DO), lambda p: (0, 0)),
        ],
        out_specs=[
            pl.BlockSpec((IB, DO, B), lambda p: (p, 0, 0)),
            pl.BlockSpec((IB, N, 2, B), lambda p: (p, 0, 0, 0)),
        ],
        out_shape=[
            jax.ShapeDtypeStruct((N, DO, B), jnp.float32),
            jax.ShapeDtypeStruct((N, N, 2, B), jnp.float32),
        ],
        compiler_params=pltpu.CompilerParams(
            dimension_semantics=("arbitrary",),
        ),
    )(at, d4, at, d4, ft, gt, g, w, b2)
    return out, d4c


def kernel(diff_vecs, agent_features, A, W, b):
    at = jnp.transpose(A, (1, 2, 0))
    d4 = jnp.transpose(diff_vecs, (1, 2, 3, 0))
    ft = jnp.transpose(agent_features, (1, 2, 0))
    b2 = b.reshape(1, DO)
    out_t, d4c = _run(at, d4, ft, W, b2)
    out = jnp.transpose(out_t, (2, 0, 1))
    diff_out = jnp.transpose(d4c, (3, 0, 1, 2))
    return (diff_out, out)


# speedup vs baseline: 8.1130x; 1.0137x over previous
import jax
import jax.numpy as jnp
from jax.experimental import pallas as pl
from jax.experimental.pallas import tpu as pltpu

B, N, D, DO = 256, 64, 16, 16
IB = 16


def _g_body(f_ref, w_ref, g_ref, gt_ref):
    w2 = w_ref[:, D:2 * D]
    for j in range(N):
        r = jnp.dot(w2, f_ref[j])
        gt_ref[j] = r
        g_ref[:, j, :] = r


def _main_body(a_ref, d4_ref, ad_ref, dd_ref, f_ref, gt_ref, g_ref, w_ref,
               b_ref, o_ref, o2_ref):
    a = a_ref[...]
    d4 = d4_ref[...]
    o2_ref[...] = d4
    dxv = d4[:, :, 0, :]
    dyv = d4[:, :, 1, :]

    dxw = jnp.sum(a * dxv, axis=1)
    dyw = jnp.sum(a * dyv, axis=1)

    g = g_ref[...]
    w1 = w_ref[:, :D]
    w30 = w_ref[:, 2 * D:2 * D + 1]
    w31 = w_ref[:, 2 * D + 1:2 * D + 2]
    bs = (N - 1.0) * jnp.transpose(b_ref[...], (1, 0))
    for r in range(IB):
        s = jnp.sum(g * a[r][None], axis=1)
        base = (N - 1.0) * jnp.dot(w1, f_ref[r])
        g_i = gt_ref[r]
        aii = ad_ref[r, r]
        vdx = dd_ref[r, r, 0]
        vdy = dd_ref[r, r, 1]
        dcon = w30 * dxw[r][None] + w31 * dyw[r][None]
        selfd = g_i + w30 * vdx[None] + w31 * vdy[None]
        out = base + bs + s + dcon - aii[None] * selfd
        o_ref[r] = jnp.maximum(out, 0.0)


@jax.jit
def _run(at, d4, ft, w, b2):
    g, gt = pl.pallas_call(
        _g_body,
        in_specs=[
            pl.BlockSpec((N, D, B), lambda: (0, 0, 0)),
            pl.BlockSpec((DO, 2 * D + 2), lambda: (0, 0)),
        ],
        out_specs=[
            pl.BlockSpec((DO, N, B), lambda: (0, 0, 0)),
            pl.BlockSpec((N, DO, B), lambda: (0, 0, 0)),
        ],
        out_shape=[
            jax.ShapeDtypeStruct((DO, N, B), jnp.float32),
            jax.ShapeDtypeStruct((N, DO, B), jnp.float32),
        ],
    )(ft, w)
    out, d4c = pl.pallas_call(
        _main_body,
        grid=(N // IB,),
        in_specs=[
            pl.BlockSpec((IB, N, B), lambda p: (p, 0, 0)),
            pl.BlockSpec((IB, N, 2, B), lambda p: (p, 0, 0, 0)),
            pl.BlockSpec((IB, IB, B), lambda p: (p, p, 0)),
            pl.BlockSpec((IB, IB, 2, B), lambda p: (p, p, 0, 0)),
            pl.BlockSpec((IB, D, B), lambda p: (p, 0, 0)),
            pl.BlockSpec((IB, DO, B), lambda p: (p, 0, 0)),
            pl.BlockSpec((DO, N, B), lambda p: (0, 0, 0)),
            pl.BlockSpec((DO, 2 * D + 2), lambda p: (0, 0)),
            pl.BlockSpec((1, DO), lambda p: (0, 0)),
        ],
        out_specs=[
            pl.BlockSpec((IB, DO, B), lambda p: (p, 0, 0)),
            pl.BlockSpec((IB, N, 2, B), lambda p: (p, 0, 0, 0)),
        ],
        out_shape=[
            jax.ShapeDtypeStruct((N, DO, B), jnp.float32),
            jax.ShapeDtypeStruct((N, N, 2, B), jnp.float32),
        ],
        compiler_params=pltpu.CompilerParams(
            dimension_semantics=("arbitrary",),
        ),
    )(at, d4, at, d4, ft, gt, g, w, b2)
    return out, d4c


def kernel(diff_vecs, agent_features, A, W, b):
    at = jnp.transpose(A, (1, 2, 0))
    d4 = jnp.transpose(diff_vecs, (1, 2, 3, 0))
    ft = jnp.transpose(agent_features, (1, 2, 0))
    b2 = b.reshape(1, DO)
    out_t, d4c = _run(at, d4, ft, W, b2)
    out = jnp.transpose(out_t, (2, 0, 1))
    diff_out = jnp.transpose(d4c, (3, 0, 1, 2))
    return (diff_out, out)


# speedup vs baseline: 9.0550x; 1.1161x over previous
import jax
import jax.numpy as jnp
from jax.experimental import pallas as pl
from jax.experimental.pallas import tpu as pltpu

B, N, D, DO = 256, 64, 16, 16
IB = 16


def _main_body(a_ref, d4_ref, ad_ref, dd_ref, f_ref, ftf_ref, w_ref,
               b_ref, o_ref, o2_ref, g_scr, gt_scr):
    p = pl.program_id(0)

    @pl.when(p == 0)
    def _():
        w2 = w_ref[:, D:2 * D]
        for j in range(N):
            r = jnp.dot(w2, ftf_ref[j])
            gt_scr[j] = r
            g_scr[:, j, :] = r

    a = a_ref[...]
    d4 = d4_ref[...]
    o2_ref[...] = d4
    dxv = d4[:, :, 0, :]
    dyv = d4[:, :, 1, :]

    dxw = jnp.sum(a * dxv, axis=1)
    dyw = jnp.sum(a * dyv, axis=1)

    g = g_scr[...]
    w1 = w_ref[:, :D]
    w30 = w_ref[:, 2 * D:2 * D + 1]
    w31 = w_ref[:, 2 * D + 1:2 * D + 2]
    bs = (N - 1.0) * jnp.transpose(b_ref[...], (1, 0))
    for r in range(IB):
        s = jnp.sum(g * a[r][None], axis=1)
        base = (N - 1.0) * jnp.dot(w1, f_ref[r])
        g_i = gt_scr[p * IB + r]
        aii = ad_ref[r, r]
        vdx = dd_ref[r, r, 0]
        vdy = dd_ref[r, r, 1]
        dcon = w30 * dxw[r][None] + w31 * dyw[r][None]
        selfd = g_i + w30 * vdx[None] + w31 * vdy[None]
        out = base + bs + s + dcon - aii[None] * selfd
        o_ref[r] = jnp.maximum(out, 0.0)


@jax.jit
def _run(at, d4, ft, w, b2):
    out, d4c = pl.pallas_call(
        _main_body,
        grid=(N // IB,),
        in_specs=[
            pl.BlockSpec((IB, N, B), lambda p: (p, 0, 0)),
            pl.BlockSpec((IB, N, 2, B), lambda p: (p, 0, 0, 0)),
            pl.BlockSpec((IB, IB, B), lambda p: (p, p, 0)),
            pl.BlockSpec((IB, IB, 2, B), lambda p: (p, p, 0, 0)),
            pl.BlockSpec((IB, D, B), lambda p: (p, 0, 0)),
            pl.BlockSpec((N, D, B), lambda p: (0, 0, 0)),
            pl.BlockSpec((DO, 2 * D + 2), lambda p: (0, 0)),
            pl.BlockSpec((1, DO), lambda p: (0, 0)),
        ],
        scratch_shapes=[
            pltpu.VMEM((DO, N, B), jnp.float32),
            pltpu.VMEM((N, DO, B), jnp.float32),
        ],
        out_specs=[
            pl.BlockSpec((IB, DO, B), lambda p: (p, 0, 0)),
            pl.BlockSpec((IB, N, 2, B), lambda p: (p, 0, 0, 0)),
        ],
        out_shape=[
            jax.ShapeDtypeStruct((N, DO, B), jnp.float32),
            jax.ShapeDtypeStruct((N, N, 2, B), jnp.float32),
        ],
        compiler_params=pltpu.CompilerParams(
            dimension_semantics=("arbitrary",),
        ),
    )(at, d4, at, d4, ft, ft, w, b2)
    return out, d4c


def kernel(diff_vecs, agent_features, A, W, b):
    at = jnp.transpose(A, (1, 2, 0))
    d4 = jnp.transpose(diff_vecs, (1, 2, 3, 0))
    ft = jnp.transpose(agent_features, (1, 2, 0))
    b2 = b.reshape(1, DO)
    out_t, d4c = _run(at, d4, ft, W, b2)
    out = jnp.transpose(out_t, (2, 0, 1))
    diff_out = jnp.transpose(d4c, (3, 0, 1, 2))
    return (diff_out, out)
